# baseline (speedup 1.0000x reference)
import jax
import jax.numpy as jnp
from jax.experimental import pallas as pl
from jax.experimental.pallas import tpu as pltpu

B, N, HID, NODE_DIM = 4, 200, 256, 8
_EPS = 1e-5


def _leaky(v):
    return jnp.where(v >= 0, v, 0.2 * v)


def _bn(v, g, b):
    mu = jnp.mean(v, axis=0, keepdims=True)
    var = jnp.mean((v - mu) ** 2, axis=0, keepdims=True)
    return (v - mu) / jnp.sqrt(var + _EPS) * g + b


def _mmT(a, w):
    return jax.lax.dot_general(
        a, w, (((1,), (1,)), ((), ())), preferred_element_type=jnp.float32,
        precision=jax.lax.Precision.HIGHEST,
    )


def _dotT(a, b):
    return jax.lax.dot_general(
        a, b, (((0,), (0,)), ((), ())), preferred_element_type=jnp.float32,
        precision=jax.lax.Precision.HIGHEST,
    )


def _fwd_kernel(sc_ref, x_ref,
                g1_ref, g1b_ref, el1_ref, el1b_ref,
                lin1_hbm, lin1b_ref, bn1g_ref, bn1b_ref,
                g2_hbm, g2b_ref, el2_hbm, el2b_ref,
                lin2a_ref, lin2ab_ref, lin2b_ref, lin2bb_ref,
                bn2g_ref, bn2b_ref,
                fc1g_hbm, fc1b_ref, fc2_ref, fc2b_ref, fc3_ref, fc3b_ref,
                out_ref,
                lin1_v, g2_v, el2_v, fc1g_v, sem1, sem2, sem3, sem4):
    cp1 = pltpu.make_async_copy(lin1_hbm, lin1_v, sem1)
    cp2 = pltpu.make_async_copy(g2_hbm, g2_v, sem2)
    cp3 = pltpu.make_async_copy(el2_hbm, el2_v, sem3)
    cp4 = pltpu.make_async_copy(fc1g_hbm, fc1g_v, sem4)
    cp1.start()
    cp2.start()
    cp3.start()
    cp4.start()

    ones = jnp.ones((N, 1), jnp.float32)
    masks = [(sc_ref[b] != 0).astype(jnp.float32) for b in range(B)]
    deg = jnp.concatenate([_dotT(m, ones) for m in masks], axis=0)

    def mpgcn(z, g, gb, el, elb):
        xl = _mmT(z, g)
        agg = jnp.concatenate(
            [_dotT(masks[b], xl[b * N:(b + 1) * N]) for b in range(B)], axis=0
        )
        return (deg * (_mmT(xl, el[:, :HID]) + elb)
                + _mmT(agg, el[:, HID:]) + gb)

    z = mpgcn(x_ref[...], g1_ref[...], g1b_ref[...],
              el1_ref[...], el1b_ref[...])
    cp1.wait()
    z = _bn(_leaky(_mmT(z, lin1_v[...]) + lin1b_ref[...]),
            bn1g_ref[...], bn1b_ref[...])
    cp2.wait()
    cp3.wait()
    z = mpgcn(z, g2_v[...], g2b_ref[...],
              el2_v[...], el2b_ref[...])
    z = _leaky(_mmT(z, lin2a_ref[...]) + lin2ab_ref[...])
    z = _leaky(_mmT(z, lin2b_ref[...]) + lin2bb_ref[...])
    z = _bn(z, bn2g_ref[...], bn2b_ref[...])
    cp4.wait()
    acc = None
    for d in range(NODE_DIM):
        zd = jnp.concatenate(
            [z[b * N:(b + 1) * N, d:d + 1] for b in range(B)], axis=1
        )
        t = _dotT(zd, fc1g_v[d])
        acc = t if acc is None else acc + t
    h = _leaky(acc + fc1b_ref[...])
    h = _leaky(_mmT(h, fc2_ref[...]) + fc2b_ref[...])
    out_ref[...] = _mmT(h, fc3_ref[...]) + fc3b_ref[...]


def kernel(sparse_connection, corr, gcn1_W, gcn1_b, edge_lin1_W, edge_lin1_b,
           lin1_W, lin1_b, bn1_gamma, bn1_beta, gcn2_W, gcn2_b, edge_lin2_W,
           edge_lin2_b, lin2a_W, lin2a_b, lin2b_W, lin2b_b, bn2_gamma,
           bn2_beta, fc1_W, fc1_b, fc2_W, fc2_b, fc3_W, fc3_b):
    row = lambda v: v.reshape(1, -1)
    args = (
        sparse_connection,
        corr.reshape(B * N, N),
        gcn1_W, row(gcn1_b),
        edge_lin1_W, row(edge_lin1_b),
        lin1_W, row(lin1_b), row(bn1_gamma), row(bn1_beta),
        gcn2_W, row(gcn2_b),
        edge_lin2_W, row(edge_lin2_b),
        lin2a_W, row(lin2a_b), lin2b_W, row(lin2b_b),
        row(bn2_gamma), row(bn2_beta),
        jnp.transpose(fc1_W.reshape(256, N, NODE_DIM), (2, 1, 0)),
        row(fc1_b), fc2_W, row(fc2_b), fc3_W, row(fc3_b),
    )
    vmem = pl.BlockSpec(memory_space=pltpu.MemorySpace.VMEM)
    hbm = pl.BlockSpec(memory_space=pltpu.MemorySpace.HBM)
    specs = [vmem] * len(args)
    for i in (6, 10, 12, 20):
        specs[i] = hbm
    return pl.pallas_call(
        _fwd_kernel,
        in_specs=specs,
        out_shape=jax.ShapeDtypeStruct((B, 2), jnp.float32),
        scratch_shapes=[
            pltpu.VMEM((HID, HID), jnp.float32),
            pltpu.VMEM((HID, HID), jnp.float32),
            pltpu.VMEM((HID, 2 * HID), jnp.float32),
            pltpu.VMEM((NODE_DIM, N, HID), jnp.float32),
            pltpu.SemaphoreType.DMA,
            pltpu.SemaphoreType.DMA,
            pltpu.SemaphoreType.DMA,
            pltpu.SemaphoreType.DMA,
        ],
    )(*args)


# speedup vs baseline: 1.7614x; 1.7614x over previous
import jax
import jax.numpy as jnp
from jax.experimental import pallas as pl
from jax.experimental.pallas import tpu as pltpu

B, N, HID, NODE_DIM = 4, 200, 256, 8
_EPS = 1e-5


def _leaky(v):
    return jnp.where(v >= 0, v, 0.2 * v)


def _bn(v, g, b):
    mu = jnp.mean(v, axis=0, keepdims=True)
    var = jnp.mean((v - mu) ** 2, axis=0, keepdims=True)
    return (v - mu) / jnp.sqrt(var + _EPS) * g + b


def _mmT(a, w):
    return jax.lax.dot_general(
        a, w, (((1,), (1,)), ((), ())), preferred_element_type=jnp.float32
    )


def _dotT(a, b):
    return jax.lax.dot_general(
        a, b, (((0,), (0,)), ((), ())), preferred_element_type=jnp.float32
    )


def _fwd_kernel(sc_ref, x_ref,
                g1_ref, g1b_ref, el1_ref, el1b_ref,
                lin1_hbm, lin1b_ref, bn1g_ref, bn1b_ref,
                g2_hbm, g2b_ref, el2_hbm, el2b_ref,
                lin2a_ref, lin2ab_ref, lin2b_ref, lin2bb_ref,
                bn2g_ref, bn2b_ref,
                fc1g_hbm, fc1b_ref, fc2_ref, fc2b_ref, fc3_ref, fc3b_ref,
                out_ref,
                lin1_v, g2_v, el2_v, fc1g_v, sem1, sem2, sem3, sem4):
    cp1 = pltpu.make_async_copy(lin1_hbm, lin1_v, sem1)
    cp2 = pltpu.make_async_copy(g2_hbm, g2_v, sem2)
    cp3 = pltpu.make_async_copy(el2_hbm, el2_v, sem3)
    cp4 = pltpu.make_async_copy(fc1g_hbm, fc1g_v, sem4)
    cp1.start()
    cp2.start()
    cp3.start()
    cp4.start()

    ones = jnp.ones((N, 1), jnp.float32)
    masks = [(sc_ref[b] != 0).astype(jnp.float32) for b in range(B)]
    deg = jnp.concatenate([_dotT(m, ones) for m in masks], axis=0)

    def mpgcn(z, g, gb, el, elb):
        xl = _mmT(z, g)
        agg = jnp.concatenate(
            [_dotT(masks[b], xl[b * N:(b + 1) * N]) for b in range(B)], axis=0
        )
        return (deg * (_mmT(xl, el[:, :HID]) + elb)
                + _mmT(agg, el[:, HID:]) + gb)

    z = mpgcn(x_ref[...], g1_ref[...], g1b_ref[...],
              el1_ref[...], el1b_ref[...])
    cp1.wait()
    z = _bn(_leaky(_mmT(z, lin1_v[...]) + lin1b_ref[...]),
            bn1g_ref[...], bn1b_ref[...])
    cp2.wait()
    cp3.wait()
    z = mpgcn(z, g2_v[...], g2b_ref[...],
              el2_v[...], el2b_ref[...])
    z = _leaky(_mmT(z, lin2a_ref[...]) + lin2ab_ref[...])
    z = _leaky(_mmT(z, lin2b_ref[...]) + lin2bb_ref[...])
    z = _bn(z, bn2g_ref[...], bn2b_ref[...])
    cp4.wait()
    acc = None
    for d in range(NODE_DIM):
        zd = jnp.concatenate(
            [z[b * N:(b + 1) * N, d:d + 1] for b in range(B)], axis=1
        )
        t = _dotT(zd, fc1g_v[d])
        acc = t if acc is None else acc + t
    h = _leaky(acc + fc1b_ref[...])
    h = _leaky(_mmT(h, fc2_ref[...]) + fc2b_ref[...])
    out_ref[...] = _mmT(h, fc3_ref[...]) + fc3b_ref[...]


def kernel(sparse_connection, corr, gcn1_W, gcn1_b, edge_lin1_W, edge_lin1_b,
           lin1_W, lin1_b, bn1_gamma, bn1_beta, gcn2_W, gcn2_b, edge_lin2_W,
           edge_lin2_b, lin2a_W, lin2a_b, lin2b_W, lin2b_b, bn2_gamma,
           bn2_beta, fc1_W, fc1_b, fc2_W, fc2_b, fc3_W, fc3_b):
    row = lambda v: v.reshape(1, -1)
    args = (
        sparse_connection,
        corr.reshape(B * N, N),
        gcn1_W, row(gcn1_b),
        edge_lin1_W, row(edge_lin1_b),
        lin1_W, row(lin1_b), row(bn1_gamma), row(bn1_beta),
        gcn2_W, row(gcn2_b),
        edge_lin2_W, row(edge_lin2_b),
        lin2a_W, row(lin2a_b), lin2b_W, row(lin2b_b),
        row(bn2_gamma), row(bn2_beta),
        jnp.transpose(fc1_W.reshape(256, N, NODE_DIM), (2, 1, 0)),
        row(fc1_b), fc2_W, row(fc2_b), fc3_W, row(fc3_b),
    )
    vmem = pl.BlockSpec(memory_space=pltpu.MemorySpace.VMEM)
    hbm = pl.BlockSpec(memory_space=pltpu.MemorySpace.HBM)
    specs = [vmem] * len(args)
    for i in (6, 10, 12, 20):
        specs[i] = hbm
    return pl.pallas_call(
        _fwd_kernel,
        in_specs=specs,
        out_shape=jax.ShapeDtypeStruct((B, 2), jnp.float32),
        scratch_shapes=[
            pltpu.VMEM((HID, HID), jnp.float32),
            pltpu.VMEM((HID, HID), jnp.float32),
            pltpu.VMEM((HID, 2 * HID), jnp.float32),
            pltpu.VMEM((NODE_DIM, N, HID), jnp.float32),
            pltpu.SemaphoreType.DMA,
            pltpu.SemaphoreType.DMA,
            pltpu.SemaphoreType.DMA,
            pltpu.SemaphoreType.DMA,
        ],
    )(*args)
